# baseline (speedup 1.0000x reference)
import functools

import jax
import jax.numpy as jnp
from jax import lax
from jax.experimental import pallas as pl
from jax.experimental.pallas import tpu as pltpu
from jax.experimental.pallas import tpu_sc as plsc

N_REL = 49
R_SIZE = 24
T_STEPS = 3
L_LAYERS = 2
N_ENT = 50000
N_TRIPLES = 320000
EMB = 128
TAU1 = 10.0
BATCH = 8

LANES = 16
NC = 2
NS = 16
CH = 128
NCHUNK = 160
TPAD = NS * NCHUNK * CH
EPAD = 53248
ROWS_SC = EPAD // NS
ZCH = ROWS_SC // CH
DUMP = N_ENT


def _tc_body(input_r_ref, emb_ref, wihT_ref, whhT_ref, b_ref, linwT_ref,
             linb_ref, wout_ref):
    rows = [emb_ref[pl.ds(input_r_ref[b], 1), :] for b in range(BATCH)]
    er = jnp.concatenate(rows, axis=0)
    last = jnp.broadcast_to(emb_ref[pl.ds(N_REL - 1, 1), :], (BATCH, EMB))
    xs = [er, er, er, last]

    def run_lstm(seq, l, d):
        W = wihT_ref[l, d]
        U = whhT_ref[l, d]
        bb = b_ref[l, d][None, :]
        h = jnp.zeros((BATCH, EMB), jnp.float32)
        c = jnp.zeros((BATCH, EMB), jnp.float32)
        hs = []
        for xt in seq:
            g = (jnp.dot(xt, W, preferred_element_type=jnp.float32)
                 + jnp.dot(h, U, preferred_element_type=jnp.float32) + bb)
            i = jax.nn.sigmoid(g[:, 0 * EMB:1 * EMB])
            f = jax.nn.sigmoid(g[:, 1 * EMB:2 * EMB])
            gg = jnp.tanh(g[:, 2 * EMB:3 * EMB])
            o = jax.nn.sigmoid(g[:, 3 * EMB:4 * EMB])
            c = f * c + i * gg
            h = o * jnp.tanh(c)
            hs.append(h)
        return hs

    lane = lax.broadcasted_iota(jnp.int32, (BATCH, EMB), 1)
    valid = lane < N_REL
    for l in range(L_LAYERS):
        hf = run_lstm(xs, l, 0)
        hb = run_lstm(xs[::-1], l, 1)[::-1]
        for t in range(T_STEPS):
            hcat = jnp.concatenate([hf[t], hb[t]], axis=1)
            lg = (jnp.dot(hcat, linwT_ref[...],
                          preferred_element_type=jnp.float32)
                  + linb_ref[...])
            z = jnp.where(valid, lg * (1.0 / TAU1), -1e30)
            m = jnp.max(z, axis=-1, keepdims=True)
            p = jnp.where(valid, jnp.exp(z - m), 0.0)
            w = p / jnp.sum(p, axis=-1, keepdims=True)
            wout_ref[t, l * BATCH:(l + 1) * BATCH, :] = w


def _tc_weights(input_r, emb, lstm_Wih, lstm_Whh, lstm_b, linear_w, linear_b):
    wihT = jnp.transpose(lstm_Wih, (0, 1, 3, 2)).astype(jnp.float32)
    whhT = jnp.transpose(lstm_Whh, (0, 1, 3, 2)).astype(jnp.float32)
    bb = lstm_b.astype(jnp.float32)
    linwT = jnp.zeros((2 * EMB, 128), jnp.float32)
    linwT = linwT.at[:, :N_REL].set(linear_w.astype(jnp.float32).T)
    linb = jnp.zeros((1, 128), jnp.float32).at[0, :N_REL].set(
        linear_b.astype(jnp.float32))
    return pl.pallas_call(
        _tc_body,
        out_shape=jax.ShapeDtypeStruct((T_STEPS, LANES, 128), jnp.float32),
        in_specs=[
            pl.BlockSpec(memory_space=pltpu.SMEM),
            pl.BlockSpec(memory_space=pltpu.VMEM),
            pl.BlockSpec(memory_space=pltpu.VMEM),
            pl.BlockSpec(memory_space=pltpu.VMEM),
            pl.BlockSpec(memory_space=pltpu.VMEM),
            pl.BlockSpec(memory_space=pltpu.VMEM),
            pl.BlockSpec(memory_space=pltpu.VMEM),
        ],
        out_specs=pl.BlockSpec(memory_space=pltpu.VMEM),
    )(input_r.astype(jnp.int32), emb.astype(jnp.float32), wihT, whhT, bb,
      linwT, linb)


def _sc_body(x_hbm, gh_hbm, rh_hbm, dh_hbm, gt_hbm, dt_hbm,
             wcomb_hbm, ws_hbm,
             out_hbm,
             acc, xsh, ssh, wsh,
             gb0, rb0, db0, gb1, rb1, db1,
             gb2, rb2, db2, gb3, rb3, db3,
             xr0, xr1, wr0, wr1, zbuf, svref, sbuf, wsv, wvb,
             semi0, semi1, semi2, semi3, semg0, semg1):
    cid = lax.axis_index("c")
    sid = lax.axis_index("s")
    zbase = sid * ROWS_SC

    pltpu.sync_copy(x_hbm.at[pl.ds(zbase, ROWS_SC)],
                    xsh.at[pl.ds(zbase, ROWS_SC)])
    pltpu.sync_copy(ws_hbm, wsv)

    @pl.when(sid == 0)
    def _():
        pltpu.sync_copy(wcomb_hbm, wvb)
        pltpu.sync_copy(wvb, wsh)

    zv = jnp.zeros((LANES,), jnp.float32)
    for j in range(CH):
        zbuf[j, :] = zv

    def zero_body(z, carry):
        pltpu.sync_copy(zbuf, acc.at[pl.ds(zbase + z * CH, CH)])
        return carry
    lax.fori_loop(0, ZCH, zero_body, 0)
    plsc.subcore_barrier()

    ibufs = ((gb0, rb0, db0, semi0), (gb1, rb1, db1, semi1),
             (gb2, rb2, db2, semi2), (gb3, rb3, db3, semi3))
    gbufs = ((xr0, wr0, semg0), (xr1, wr1, semg1))

    def step_body(t, tot):
        sv = jnp.zeros((LANES,), jnp.float32)
        for d, (g_hbm, d_hbm) in enumerate(((gh_hbm, dh_hbm),
                                            (gt_hbm, dt_hbm))):
            roff = (jnp.zeros((LANES,), jnp.int32)
                    + t * (2 * R_SIZE) + d * R_SIZE)

            def issue_idx(g, par):
                gb, rb, db, semi = ibufs[par]
                pltpu.async_copy(g_hbm.at[sid, g], gb, semi)
                pltpu.async_copy(rh_hbm.at[sid, g], rb, semi)
                pltpu.async_copy(d_hbm.at[sid, g], db, semi)

            def wait_idx(g, par):
                gb, rb, db, semi = ibufs[par]
                pltpu.make_async_copy(g_hbm.at[sid, g], gb, semi).wait()
                pltpu.make_async_copy(rh_hbm.at[sid, g], rb, semi).wait()
                pltpu.make_async_copy(d_hbm.at[sid, g], db, semi).wait()
                for k in range(CH // LANES):
                    sl = pl.ds(k * LANES, LANES)
                    rb[sl] = rb[sl] + roff

            def issue_gather(ipar, par):
                gb, rb, db, _ = ibufs[ipar]
                xr, wr, semg = gbufs[par]
                pltpu.async_copy(xsh.at[gb], xr, semg)
                pltpu.async_copy(wsh.at[rb], wr, semg)

            def finish(ipar, par, s_carry):
                gb, rb, db, _ = ibufs[ipar]
                xr, wr, semg = gbufs[par]
                pltpu.make_async_copy(xsh.at[gb], xr, semg).wait()
                pltpu.make_async_copy(wsh.at[rb], wr, semg).wait()
                for j in range(CH):
                    v = xr[j, :] * wr[j, :]
                    xr[j, :] = v
                    s_carry = s_carry + v
                pltpu.sync_copy(xr, acc.at[db], add=True)
                return s_carry

            for g in range(4):
                issue_idx(g, g)
            wait_idx(0, 0)
            issue_gather(0, 0)

            def quad_body(i, s_carry):
                for p in range(4):
                    g = 4 * i + p

                    @pl.when(g + 1 < NCHUNK)
                    def _():
                        wait_idx(g + 1, (p + 1) % 4)
                        issue_gather((p + 1) % 4, (p + 1) % 2)
                    s_carry = finish(p, p % 2, s_carry)

                    @pl.when(g + 4 < NCHUNK)
                    def _():
                        issue_idx(g + 4, p)
                return s_carry
            sv = lax.fori_loop(0, NCHUNK // 4, quad_body, sv)

        svref[...] = sv
        pltpu.sync_copy(svref, ssh.at[sid])
        plsc.subcore_barrier()
        pltpu.sync_copy(ssh, sbuf)
        S = sbuf[0, :]
        for i in range(1, NS):
            S = S + sbuf[i, :]
        ws = wsv[t, :]
        S = S + ws * tot
        Sc = jnp.maximum(S, jnp.float32(1e-7))
        inv = jnp.float32(1.0) / Sc

        last = t == T_STEPS - 1

        def norm_body(z, carry):
            off = zbase + z * CH
            pltpu.sync_copy(acc.at[pl.ds(off, CH)], xr0)
            pltpu.sync_copy(xsh.at[pl.ds(off, CH)], wr0)
            for j in range(CH):
                xr0[j, :] = (xr0[j, :] + ws * wr0[j, :]) * inv

            @pl.when(jnp.logical_and(last, cid == 0))
            def _():
                pltpu.sync_copy(xr0, out_hbm.at[pl.ds(off, CH)])

            @pl.when(jnp.logical_not(last))
            def _():
                pltpu.sync_copy(xr0, xsh.at[pl.ds(off, CH)])
                pltpu.sync_copy(zbuf, acc.at[pl.ds(off, CH)])
            return carry
        lax.fori_loop(0, ZCH, norm_body, 0)
        plsc.subcore_barrier()
        return S * inv

    lax.fori_loop(0, T_STEPS, step_body, jnp.ones((LANES,), jnp.float32))


@functools.lru_cache(maxsize=1)
def _sc_kernel():
    mesh = plsc.VectorSubcoreMesh(core_axis_name="c", subcore_axis_name="s",
                                  num_cores=NC, num_subcores=NS)
    params = pltpu.CompilerParams(use_tc_tiling_on_sc=False)
    return pl.kernel(
        _sc_body,
        out_type=[jax.ShapeDtypeStruct((EPAD, LANES), jnp.float32)],
        mesh=mesh,
        scratch_types=[
            pltpu.VMEM_SHARED((EPAD, LANES), jnp.float32),
            pltpu.VMEM_SHARED((EPAD, LANES), jnp.float32),
            pltpu.VMEM_SHARED((NS, LANES), jnp.float32),
            pltpu.VMEM_SHARED((6 * R_SIZE, LANES), jnp.float32),
        ] + [pltpu.VMEM((CH,), jnp.int32)] * 12
          + [pltpu.VMEM((CH, LANES), jnp.float32)] * 4
          + [
            pltpu.VMEM((CH, LANES), jnp.float32),
            pltpu.VMEM((LANES,), jnp.float32),
            pltpu.VMEM((NS, LANES), jnp.float32),
            pltpu.VMEM((T_STEPS, LANES), jnp.float32),
            pltpu.VMEM((6 * R_SIZE, LANES), jnp.float32),
            pltpu.SemaphoreType.DMA,
            pltpu.SemaphoreType.DMA,
            pltpu.SemaphoreType.DMA,
            pltpu.SemaphoreType.DMA,
            pltpu.SemaphoreType.DMA,
            pltpu.SemaphoreType.DMA,
        ],
        compiler_params=params,
    )


def _pad_chunks(a, fill):
    a = a.astype(jnp.int32)
    pad = TPAD - N_TRIPLES
    a = jnp.concatenate([a, jnp.full((pad,), fill, jnp.int32)])
    return a.reshape(NS, NCHUNK, CH)


def kernel(input_x, input_r, e2triple, triple2e, r2triple, emb,
           lstm_Wih, lstm_Whh, lstm_b, linear_w, linear_b):
    wout = _tc_weights(input_r, emb, lstm_Wih, lstm_Whh, lstm_b,
                       linear_w, linear_b)
    whtab = jnp.transpose(wout[:, :, :R_SIZE], (0, 2, 1))
    wttab = jnp.transpose(wout[:, :, R_SIZE:2 * R_SIZE], (0, 2, 1))
    wcomb = jnp.stack([whtab, wttab], axis=1).reshape(6 * R_SIZE, LANES)
    wself = wout[:, :, 2 * R_SIZE]

    head = e2triple[0]
    ent2 = e2triple[2]
    tail = triple2e[1]
    rel = r2triple[0]
    gh = _pad_chunks(head, DUMP)
    dh = _pad_chunks(tail, DUMP)
    gt = _pad_chunks(ent2, DUMP)
    dt = _pad_chunks(head, DUMP)
    rh = _pad_chunks(rel, 0)

    bidx = jnp.arange(BATCH)
    x0 = jnp.zeros((EPAD, LANES), jnp.float32)
    x0 = x0.at[input_x, bidx].set(1.0).at[input_x, BATCH + bidx].set(1.0)

    xf, = _sc_kernel()(x0, gh, rh, dh, gt, dt, wcomb, wself)
    out = xf[:N_ENT, :BATCH] + xf[:N_ENT, BATCH:]
    return out.T


# speedup vs baseline: 1.6207x; 1.6207x over previous
import functools

import jax
import jax.numpy as jnp
from jax import lax
from jax.experimental import pallas as pl
from jax.experimental.pallas import tpu as pltpu
from jax.experimental.pallas import tpu_sc as plsc

N_REL = 49
R_SIZE = 24
T_STEPS = 3
L_LAYERS = 2
N_ENT = 50000
N_TRIPLES = 320000
EMB = 128
TAU1 = 10.0
BATCH = 8

LANES = 16
NC = 2
NS = 16
NW = NC * NS
CH = 128
NCHUNK = 80
TPAD = NW * NCHUNK * CH
EPAD = 53248
ROWS_SC = EPAD // NS
ZCH = ROWS_SC // CH
ROWS_B = EPAD // NW
BCH = ROWS_B // CH
DUMP = N_ENT


def _tc_body(input_r_ref, emb_ref, wihT_ref, whhT_ref, b_ref, linwT_ref,
             linb_ref, wout_ref):
    rows = [emb_ref[pl.ds(input_r_ref[b], 1), :] for b in range(BATCH)]
    er = jnp.concatenate(rows, axis=0)
    last = jnp.broadcast_to(emb_ref[pl.ds(N_REL - 1, 1), :], (BATCH, EMB))
    xs = [er, er, er, last]

    def run_lstm(seq, l, d):
        W = wihT_ref[l, d]
        U = whhT_ref[l, d]
        bb = b_ref[l, d][None, :]
        h = jnp.zeros((BATCH, EMB), jnp.float32)
        c = jnp.zeros((BATCH, EMB), jnp.float32)
        hs = []
        for xt in seq:
            g = (jnp.dot(xt, W, preferred_element_type=jnp.float32)
                 + jnp.dot(h, U, preferred_element_type=jnp.float32) + bb)
            i = jax.nn.sigmoid(g[:, 0 * EMB:1 * EMB])
            f = jax.nn.sigmoid(g[:, 1 * EMB:2 * EMB])
            gg = jnp.tanh(g[:, 2 * EMB:3 * EMB])
            o = jax.nn.sigmoid(g[:, 3 * EMB:4 * EMB])
            c = f * c + i * gg
            h = o * jnp.tanh(c)
            hs.append(h)
        return hs

    lane = lax.broadcasted_iota(jnp.int32, (BATCH, EMB), 1)
    valid = lane < N_REL
    for l in range(L_LAYERS):
        hf = run_lstm(xs, l, 0)
        hb = run_lstm(xs[::-1], l, 1)[::-1]
        for t in range(T_STEPS):
            hcat = jnp.concatenate([hf[t], hb[t]], axis=1)
            lg = (jnp.dot(hcat, linwT_ref[...],
                          preferred_element_type=jnp.float32)
                  + linb_ref[...])
            z = jnp.where(valid, lg * (1.0 / TAU1), -1e30)
            m = jnp.max(z, axis=-1, keepdims=True)
            p = jnp.where(valid, jnp.exp(z - m), 0.0)
            w = p / jnp.sum(p, axis=-1, keepdims=True)
            wout_ref[t, l * BATCH:(l + 1) * BATCH, :] = w


def _tc_weights(input_r, emb, lstm_Wih, lstm_Whh, lstm_b, linear_w, linear_b):
    wihT = jnp.transpose(lstm_Wih, (0, 1, 3, 2)).astype(jnp.float32)
    whhT = jnp.transpose(lstm_Whh, (0, 1, 3, 2)).astype(jnp.float32)
    bb = lstm_b.astype(jnp.float32)
    linwT = jnp.zeros((2 * EMB, 128), jnp.float32)
    linwT = linwT.at[:, :N_REL].set(linear_w.astype(jnp.float32).T)
    linb = jnp.zeros((1, 128), jnp.float32).at[0, :N_REL].set(
        linear_b.astype(jnp.float32))
    return pl.pallas_call(
        _tc_body,
        out_shape=jax.ShapeDtypeStruct((T_STEPS, LANES, 128), jnp.float32),
        in_specs=[
            pl.BlockSpec(memory_space=pltpu.SMEM),
            pl.BlockSpec(memory_space=pltpu.VMEM),
            pl.BlockSpec(memory_space=pltpu.VMEM),
            pl.BlockSpec(memory_space=pltpu.VMEM),
            pl.BlockSpec(memory_space=pltpu.VMEM),
            pl.BlockSpec(memory_space=pltpu.VMEM),
            pl.BlockSpec(memory_space=pltpu.VMEM),
        ],
        out_specs=pl.BlockSpec(memory_space=pltpu.VMEM),
    )(input_r.astype(jnp.int32), emb.astype(jnp.float32), wihT, whhT, bb,
      linwT, linb)


def _phase_a_body(x_hbm, z_hbm, gh_hbm, rh_hbm, dh_hbm, gt_hbm, rt_hbm,
                  dt_hbm, wh_hbm, wt_hbm, part_hbm, sums_hbm,
                  acc, xsh, wsh0, wsh1,
                  gb0, rb0, db0, gb1, rb1, db1,
                  gb2, rb2, db2, gb3, rb3, db3,
                  xr0, xr1, wr0, wr1, svref, wvb,
                  semi0, semi1, semi2, semi3, semg0, semg1):
    cid = lax.axis_index("c")
    sid = lax.axis_index("s")
    wid = cid * NS + sid
    zbase = sid * ROWS_SC

    pltpu.sync_copy(x_hbm.at[pl.ds(zbase, ROWS_SC)],
                    xsh.at[pl.ds(zbase, ROWS_SC)])
    pltpu.sync_copy(z_hbm, acc.at[pl.ds(zbase, ROWS_SC)])

    @pl.when(sid == 0)
    def _():
        pltpu.sync_copy(wh_hbm, wvb)
        pltpu.sync_copy(wvb, wsh0)
        pltpu.sync_copy(wt_hbm, wvb)
        pltpu.sync_copy(wvb, wsh1)
    plsc.subcore_barrier()

    sv = jnp.zeros((LANES,), jnp.float32)
    ibufs = ((gb0, rb0, db0, semi0), (gb1, rb1, db1, semi1),
             (gb2, rb2, db2, semi2), (gb3, rb3, db3, semi3))
    gbufs = ((xr0, wr0, semg0), (xr1, wr1, semg1))
    for (g_hbm, d_hbm, wsh) in ((gh_hbm, dh_hbm, wsh0),
                                (gt_hbm, dt_hbm, wsh1)):
        def issue_idx(g, par):
            gb, rb, db, semi = ibufs[par]
            pltpu.async_copy(g_hbm.at[wid, g], gb, semi)
            pltpu.async_copy(rh_hbm.at[wid, g], rb, semi)
            pltpu.async_copy(d_hbm.at[wid, g], db, semi)

        def wait_idx(g, par):
            gb, rb, db, semi = ibufs[par]
            pltpu.make_async_copy(g_hbm.at[wid, g], gb, semi).wait()
            pltpu.make_async_copy(rh_hbm.at[wid, g], rb, semi).wait()
            pltpu.make_async_copy(d_hbm.at[wid, g], db, semi).wait()

        def issue_gather(ipar, par):
            gb, rb, db, _ = ibufs[ipar]
            xr, wr, semg = gbufs[par]
            pltpu.async_copy(xsh.at[gb], xr, semg)
            pltpu.async_copy(wsh.at[rb], wr, semg)

        def finish(ipar, par, s_carry):
            gb, rb, db, _ = ibufs[ipar]
            xr, wr, semg = gbufs[par]
            pltpu.make_async_copy(xsh.at[gb], xr, semg).wait()
            pltpu.make_async_copy(wsh.at[rb], wr, semg).wait()
            for j in range(CH):
                v = xr[j, :] * wr[j, :]
                xr[j, :] = v
                s_carry = s_carry + v
            pltpu.sync_copy(xr, acc.at[db], add=True)
            return s_carry

        for g in range(4):
            issue_idx(g, g)
        wait_idx(0, 0)
        issue_gather(0, 0)

        def quad_body(i, s_carry):
            for p in range(4):
                g = 4 * i + p

                @pl.when(g + 1 < NCHUNK)
                def _():
                    wait_idx(g + 1, (p + 1) % 4)
                    issue_gather((p + 1) % 4, (p + 1) % 2)
                s_carry = finish(p, p % 2, s_carry)

                @pl.when(g + 4 < NCHUNK)
                def _():
                    issue_idx(g + 4, p)
            return s_carry
        sv = lax.fori_loop(0, NCHUNK // 4, quad_body, sv)

    svref[...] = sv
    pltpu.sync_copy(svref, sums_hbm.at[wid])
    plsc.subcore_barrier()

    pltpu.sync_copy(acc.at[pl.ds(zbase, ROWS_SC)],
                    part_hbm.at[cid, pl.ds(zbase, ROWS_SC)])


def _phase_b_body(part_hbm, x_hbm, sums_hbm, wself_hbm, ptot_hbm,
             xn_hbm, tot_hbm,
             sbuf, wbuf, tbuf, p0buf, p1buf, xbuf, oref, sem):
    cid = lax.axis_index("c")
    sid = lax.axis_index("s")
    wid = cid * NS + sid
    base = wid * ROWS_B

    pltpu.async_copy(part_hbm.at[0, pl.ds(base, ROWS_B)], p0buf, sem)
    pltpu.async_copy(part_hbm.at[1, pl.ds(base, ROWS_B)], p1buf, sem)
    pltpu.async_copy(x_hbm.at[pl.ds(base, ROWS_B)], xbuf, sem)

    pltpu.sync_copy(sums_hbm, sbuf)
    pltpu.sync_copy(wself_hbm, wbuf)
    pltpu.sync_copy(ptot_hbm, tbuf)

    S = sbuf[0, :]
    for i in range(1, NW):
        S = S + sbuf[i, :]
    ws = wbuf[...]
    S = S + ws * tbuf[...]
    Sc = jnp.maximum(S, jnp.float32(1e-7))
    inv = jnp.float32(1.0) / Sc

    @pl.when(wid == 0)
    def _():
        oref[...] = S * inv
        pltpu.sync_copy(oref, tot_hbm)

    pltpu.make_async_copy(part_hbm.at[0, pl.ds(base, ROWS_B)], p0buf,
                          sem).wait()
    pltpu.make_async_copy(part_hbm.at[1, pl.ds(base, ROWS_B)], p1buf,
                          sem).wait()
    pltpu.make_async_copy(x_hbm.at[pl.ds(base, ROWS_B)], xbuf, sem).wait()

    def chunk_body(z, carry):
        b2 = z * CH
        for j in range(CH):
            p0buf[b2 + j, :] = (p0buf[b2 + j, :] + p1buf[b2 + j, :]
                                + ws * xbuf[b2 + j, :]) * inv
        return carry
    lax.fori_loop(0, BCH, chunk_body, 0)
    pltpu.sync_copy(p0buf, xn_hbm.at[pl.ds(base, ROWS_B)])


@functools.lru_cache(maxsize=1)
def _sc_kernels():
    mesh = plsc.VectorSubcoreMesh(core_axis_name="c", subcore_axis_name="s",
                                  num_cores=NC, num_subcores=NS)
    params = pltpu.CompilerParams(use_tc_tiling_on_sc=False)
    phase_a = pl.kernel(
        _phase_a_body,
        out_type=[
            jax.ShapeDtypeStruct((NC, EPAD, LANES), jnp.float32),
            jax.ShapeDtypeStruct((NW, LANES), jnp.float32),
        ],
        mesh=mesh,
        scratch_types=[
            pltpu.VMEM_SHARED((EPAD, LANES), jnp.float32),
            pltpu.VMEM_SHARED((EPAD, LANES), jnp.float32),
            pltpu.VMEM_SHARED((R_SIZE, LANES), jnp.float32),
            pltpu.VMEM_SHARED((R_SIZE, LANES), jnp.float32),
        ] + [pltpu.VMEM((CH,), jnp.int32)] * 12
          + [pltpu.VMEM((CH, LANES), jnp.float32)] * 4
          + [
            pltpu.VMEM((LANES,), jnp.float32),
            pltpu.VMEM((R_SIZE, LANES), jnp.float32),
            pltpu.SemaphoreType.DMA,
            pltpu.SemaphoreType.DMA,
            pltpu.SemaphoreType.DMA,
            pltpu.SemaphoreType.DMA,
            pltpu.SemaphoreType.DMA,
            pltpu.SemaphoreType.DMA,
        ],
        compiler_params=params,
    )
    phase_b = pl.kernel(
        _phase_b_body,
        out_type=[
            jax.ShapeDtypeStruct((EPAD, LANES), jnp.float32),
            jax.ShapeDtypeStruct((LANES,), jnp.float32),
        ],
        mesh=mesh,
        scratch_types=[
            pltpu.VMEM((NW, LANES), jnp.float32),
            pltpu.VMEM((LANES,), jnp.float32),
            pltpu.VMEM((LANES,), jnp.float32),
            pltpu.VMEM((ROWS_B, LANES), jnp.float32),
            pltpu.VMEM((ROWS_B, LANES), jnp.float32),
            pltpu.VMEM((ROWS_B, LANES), jnp.float32),
            pltpu.VMEM((LANES,), jnp.float32),
            pltpu.SemaphoreType.DMA,
        ],
        compiler_params=params,
    )
    return phase_a, phase_b


def _pad_chunks(a, fill):
    a = a.astype(jnp.int32)
    pad = TPAD - N_TRIPLES
    a = jnp.concatenate([a, jnp.full((pad,), fill, jnp.int32)])
    return a.reshape(NW, NCHUNK, CH)


def kernel(input_x, input_r, e2triple, triple2e, r2triple, emb,
           lstm_Wih, lstm_Whh, lstm_b, linear_w, linear_b):
    wout = _tc_weights(input_r, emb, lstm_Wih, lstm_Whh, lstm_b,
                       linear_w, linear_b)
    whtab = jnp.transpose(wout[:, :, :R_SIZE], (0, 2, 1))
    wttab = jnp.transpose(wout[:, :, R_SIZE:2 * R_SIZE], (0, 2, 1))
    wself = wout[:, :, 2 * R_SIZE]

    head = e2triple[0]
    ent2 = e2triple[2]
    tail = triple2e[1]
    rel = r2triple[0]
    gh = _pad_chunks(head, DUMP)
    dh = _pad_chunks(tail, DUMP)
    gt = _pad_chunks(ent2, DUMP)
    dt = _pad_chunks(head, DUMP)
    rh = _pad_chunks(rel, 0)

    bidx = jnp.arange(BATCH)
    x = jnp.zeros((EPAD, LANES), jnp.float32)
    x = x.at[input_x, bidx].set(1.0).at[input_x, BATCH + bidx].set(1.0)
    tot = jnp.ones((LANES,), jnp.float32)

    zrows = jnp.zeros((ROWS_SC, LANES), jnp.float32)
    phase_a, phase_b = _sc_kernels()
    for t in range(T_STEPS):
        part, sums = phase_a(x, zrows, gh, rh, dh, gt, rh, dt,
                             whtab[t], wttab[t])
        x, tot = phase_b(part, x, sums, wself[t], tot)

    out = x[:N_ENT, :BATCH] + x[:N_ENT, BATCH:]
    return out.T
